# Initial kernel scaffold; baseline (speedup 1.0000x reference)
import jax
import jax.numpy as jnp
from jax.experimental import pallas as pl


def kernel(x, edge_index, batch, W1, b1, W2, b2, W3, b3, g1, be1, g2, be2, gW1, gb1, gW2, gb2, mW1, mb1, mW2, mb2):
    raise NotImplementedError("write your pallas kernel here")


# speedup vs baseline: 17.9474x; 17.9474x over previous
import functools

import jax
import jax.numpy as jnp
from jax import lax
from jax.experimental import pallas as pl
from jax.experimental.pallas import tpu as pltpu
from jax.experimental.pallas import tpu_sc as plsc

N = 10000
E = 320000
D = 128
G = 16

NC = 2
NS = 16
NW = NC * NS
EPW = E // NW
CHUNK = 125
NCHUNK = EPW // CHUNK
ROWS_PER_TILE = N // NS

_mesh = plsc.VectorSubcoreMesh(core_axis_name="c", subcore_axis_name="s")


def _deg_body(dst_hbm, out_hbm, dst_v, ones_v, zbuf, acc):
    cid = lax.axis_index("c")
    sid = lax.axis_index("s")
    w = cid * NS + sid

    def zinit(i, _):
        zbuf[pl.ds(i * 16, 16)] = jnp.zeros((16,), jnp.float32)
        return 0

    lax.fori_loop(0, N // 16, zinit, 0)

    @pl.when(sid == 0)
    def _():
        pltpu.sync_copy(zbuf, acc)

    def oinit(i, _):
        ones_v[pl.ds(i * 16, 16)] = jnp.ones((16,), jnp.float32)
        return 0

    lax.fori_loop(0, CHUNK // 16 + 1, oinit, 0)
    plsc.subcore_barrier()

    pltpu.sync_copy(dst_hbm.at[w], dst_v)

    def body(j, _):
        pltpu.sync_copy(ones_v.at[pl.ds(0, CHUNK)], acc.at[dst_v.at[j]],
                        add=True)
        return 0

    lax.fori_loop(0, NCHUNK, body, 0)
    plsc.subcore_barrier()

    @pl.when(sid == 0)
    def _():
        pltpu.sync_copy(acc, out_hbm.at[cid])


_deg_call = pl.kernel(
    _deg_body,
    out_type=jax.ShapeDtypeStruct((NC, N), jnp.float32),
    mesh=_mesh,
    compiler_params=pltpu.CompilerParams(use_tc_tiling_on_sc=False),
    scratch_types=[
        pltpu.VMEM((NCHUNK, CHUNK), jnp.int32),
        pltpu.VMEM((CHUNK + 16 - CHUNK % 16,), jnp.float32),
        pltpu.VMEM((N,), jnp.float32),
        pltpu.VMEM_SHARED((N,), jnp.float32),
    ],
)


CD = D // NC
EPT = E // NS
NCH = EPT // CHUNK


def _edge_body(hp_hbm, src_hbm, dst_hbm, out_hbm,
               src_v, dst_v, buf0, buf1, zbuf, acc, sem0, sem1):
    cid = lax.axis_index("c")
    sid = lax.axis_index("s")

    def zrow(r, _):
        for c in range(CD // 16):
            zbuf[r, pl.ds(c * 16, 16)] = jnp.zeros((16,), jnp.float32)
        return 0

    lax.fori_loop(0, CHUNK, zrow, 0)
    base = sid * ROWS_PER_TILE
    for k in range(ROWS_PER_TILE // CHUNK):
        pltpu.sync_copy(zbuf, acc.at[pl.ds(base + k * CHUNK, CHUNK)])
    plsc.subcore_barrier()

    pltpu.sync_copy(src_hbm.at[sid], src_v)
    pltpu.sync_copy(dst_hbm.at[sid], dst_v)
    hpc = hp_hbm.at[cid]

    pltpu.async_copy(hpc.at[src_v.at[0]], buf0, sem0)
    pltpu.async_copy(hpc.at[src_v.at[1]], buf1, sem1)

    def body(jj, _):
        for b, (buf, sem) in enumerate(((buf0, sem0), (buf1, sem1))):
            j = jj * 2 + b
            pltpu.make_async_copy(hpc.at[src_v.at[j]], buf, sem).wait()
            pltpu.sync_copy(buf, acc.at[dst_v.at[j]], add=True)

            @pl.when(j + 2 < NCH)
            def _():
                pltpu.async_copy(hpc.at[src_v.at[j + 2]], buf, sem)

        return 0

    lax.fori_loop(0, NCH // 2, body, 0)
    plsc.subcore_barrier()

    for k in range(ROWS_PER_TILE // CHUNK):
        r0 = base + k * CHUNK
        pltpu.sync_copy(acc.at[pl.ds(r0, CHUNK)], buf0)
        pltpu.sync_copy(buf0, out_hbm.at[cid].at[pl.ds(r0, CHUNK)])


_edge_call = pl.kernel(
    _edge_body,
    out_type=jax.ShapeDtypeStruct((NC, N, CD), jnp.float32),
    mesh=_mesh,
    compiler_params=pltpu.CompilerParams(use_tc_tiling_on_sc=False),
    scratch_types=[
        pltpu.VMEM((NCH, CHUNK), jnp.int32),
        pltpu.VMEM((NCH, CHUNK), jnp.int32),
        pltpu.VMEM((CHUNK, CD), jnp.float32),
        pltpu.VMEM((CHUNK, CD), jnp.float32),
        pltpu.VMEM((CHUNK, CD), jnp.float32),
        pltpu.VMEM_SHARED((N, CD), jnp.float32),
        pltpu.SemaphoreType.DMA,
        pltpu.SemaphoreType.DMA,
    ],
)


RB = 1000
NBLK = N // RB


def _k0_body(degp_ref, x_ref, w1_ref, dinv_ref, hp_ref):
    deg = jnp.sum(degp_ref[:, 0, 0, :], axis=0) + 1.0
    dinv = lax.rsqrt(deg)
    dinv_ref[0, 0, :] = dinv
    h = jnp.dot(x_ref[...], w1_ref[...], preferred_element_type=jnp.float32)
    hp = h * dinv[:, None]
    hp_ref[0] = hp[:, :CD]
    hp_ref[1] = hp[:, CD:]


def _k0(deg_parts, x, W1):
    return pl.pallas_call(
        _k0_body,
        grid=(NBLK,),
        in_specs=[
            pl.BlockSpec((NC, 1, 1, RB), lambda i: (0, i, 0, 0)),
            pl.BlockSpec((RB, D), lambda i: (i, 0)),
            pl.BlockSpec((D, D), lambda i: (0, 0)),
        ],
        out_specs=[
            pl.BlockSpec((1, 1, RB), lambda i: (i, 0, 0)),
            pl.BlockSpec((NC, RB, CD), lambda i: (0, i, 0)),
        ],
        out_shape=[
            jax.ShapeDtypeStruct((NBLK, 1, RB), jnp.float32),
            jax.ShapeDtypeStruct((NC, N, CD), jnp.float32),
        ],
    )(deg_parts, x, W1)


def _kep_body(pacc_ref, hp_ref, dinv_ref, b_ref, agg_ref, st_ref):
    i = pl.program_id(0)
    s = jnp.concatenate([pacc_ref[0] + hp_ref[0], pacc_ref[1] + hp_ref[1]],
                        axis=1)
    agg = s * dinv_ref[0, 0, :][:, None] + b_ref[...][None, :]
    agg_ref[...] = agg
    st = jnp.stack([jnp.sum(agg, axis=0), jnp.sum(agg * agg, axis=0)])

    @pl.when(i == 0)
    def _():
        st_ref[...] = st

    @pl.when(i != 0)
    def _():
        st_ref[...] += st


def _kep(pacc, hp, dinv, b):
    return pl.pallas_call(
        _kep_body,
        grid=(NBLK,),
        in_specs=[
            pl.BlockSpec((NC, RB, CD), lambda i: (0, i, 0)),
            pl.BlockSpec((NC, RB, CD), lambda i: (0, i, 0)),
            pl.BlockSpec((1, 1, RB), lambda i: (i, 0, 0)),
            pl.BlockSpec((D,), lambda i: (0,)),
        ],
        out_specs=[
            pl.BlockSpec((RB, D), lambda i: (i, 0)),
            pl.BlockSpec((2, D), lambda i: (0, 0)),
        ],
        out_shape=[
            jax.ShapeDtypeStruct((N, D), jnp.float32),
            jax.ShapeDtypeStruct((2, D), jnp.float32),
        ],
    )(pacc, hp, dinv, b)


def _kmm_body(agg_ref, st_ref, g_ref, be_ref, dinv_ref, w_ref, hp_ref):
    st = st_ref[...]
    mean = st[0] / N
    var = st[1] / N - mean * mean
    xn = (agg_ref[...] - mean[None, :]) * lax.rsqrt(var + 1e-5)[None, :]
    h = jnp.maximum(xn * g_ref[...][None, :] + be_ref[...][None, :], 0.0)
    hw = jnp.dot(h, w_ref[...], preferred_element_type=jnp.float32)
    hp = hw * dinv_ref[0, 0, :][:, None]
    hp_ref[0] = hp[:, :CD]
    hp_ref[1] = hp[:, CD:]


def _kmm(agg, st, g, be, dinv, W):
    return pl.pallas_call(
        _kmm_body,
        grid=(NBLK,),
        in_specs=[
            pl.BlockSpec((RB, D), lambda i: (i, 0)),
            pl.BlockSpec((2, D), lambda i: (0, 0)),
            pl.BlockSpec((D,), lambda i: (0,)),
            pl.BlockSpec((D,), lambda i: (0,)),
            pl.BlockSpec((1, 1, RB), lambda i: (i, 0, 0)),
            pl.BlockSpec((D, D), lambda i: (0, 0)),
        ],
        out_specs=pl.BlockSpec((NC, RB, CD), lambda i: (0, i, 0)),
        out_shape=jax.ShapeDtypeStruct((NC, N, CD), jnp.float32),
    )(agg, st, g, be, dinv, W)


def _kep3_body(pacc_ref, hp_ref, dinv_ref, b_ref, gw1_ref, gb1_ref,
               gw2_ref, gb2_ref, batch_ref, h3_ref, gate_ref, mx_ref):
    i = pl.program_id(0)
    s = jnp.concatenate([pacc_ref[0] + hp_ref[0], pacc_ref[1] + hp_ref[1]],
                        axis=1)
    h3 = s * dinv_ref[0, 0, :][:, None] + b_ref[...][None, :]
    h3_ref[...] = h3
    gmid = jnp.maximum(
        jnp.dot(h3, gw1_ref[...], preferred_element_type=jnp.float32)
        + gb1_ref[...][None, :], 0.0)
    gate = (jnp.dot(gmid, gw2_ref[...], preferred_element_type=jnp.float32)
            + gb2_ref[...][None, :])[:, 0]
    gate_ref[0, 0, :] = gate
    onehot = batch_ref[0, 0, :][:, None] == lax.broadcasted_iota(
        jnp.int32, (1, G), 1)
    blkmax = jnp.max(jnp.where(onehot, gate[:, None], -jnp.inf), axis=0)

    @pl.when(i == 0)
    def _():
        mx_ref[...] = blkmax

    @pl.when(i != 0)
    def _():
        mx_ref[...] = jnp.maximum(mx_ref[...], blkmax)


def _kep3(pacc, hp, dinv, b, gW1, gb1, gW2, gb2, batch):
    return pl.pallas_call(
        _kep3_body,
        grid=(NBLK,),
        in_specs=[
            pl.BlockSpec((NC, RB, CD), lambda i: (0, i, 0)),
            pl.BlockSpec((NC, RB, CD), lambda i: (0, i, 0)),
            pl.BlockSpec((1, 1, RB), lambda i: (i, 0, 0)),
            pl.BlockSpec((D,), lambda i: (0,)),
            pl.BlockSpec((D, D), lambda i: (0, 0)),
            pl.BlockSpec((D,), lambda i: (0,)),
            pl.BlockSpec((D, 1), lambda i: (0, 0)),
            pl.BlockSpec((1,), lambda i: (0,)),
            pl.BlockSpec((1, 1, RB), lambda i: (i, 0, 0)),
        ],
        out_specs=[
            pl.BlockSpec((RB, D), lambda i: (i, 0)),
            pl.BlockSpec((1, 1, RB), lambda i: (i, 0, 0)),
            pl.BlockSpec((G,), lambda i: (0,)),
        ],
        out_shape=[
            jax.ShapeDtypeStruct((N, D), jnp.float32),
            jax.ShapeDtypeStruct((NBLK, 1, RB), jnp.float32),
            jax.ShapeDtypeStruct((G,), jnp.float32),
        ],
    )(pacc, hp, dinv, b, gW1, gb1, gW2, gb2, batch)


def _kpool_body(h3_ref, gate_ref, batch_ref, mxin_ref, den_ref, ew_ref):
    i = pl.program_id(0)
    mxin = mxin_ref[...]
    mx = jnp.where(jnp.isfinite(mxin), mxin, 0.0)
    onehot = (batch_ref[0, 0, :][:, None] == lax.broadcasted_iota(
        jnp.int32, (1, G), 1)).astype(jnp.float32)
    mxb = jnp.dot(onehot, mx[:, None],
                  preferred_element_type=jnp.float32)[:, 0]
    e = jnp.exp(gate_ref[0, 0, :] - mxb)
    den = jnp.sum(onehot * e[:, None], axis=0)
    ew = lax.dot_general(onehot, h3_ref[...] * e[:, None],
                         (((0,), (0,)), ((), ())),
                         preferred_element_type=jnp.float32)

    @pl.when(i == 0)
    def _():
        den_ref[...] = den
        ew_ref[...] = ew

    @pl.when(i != 0)
    def _():
        den_ref[...] += den
        ew_ref[...] += ew


def _kpool(h3, gate, batch, mx):
    return pl.pallas_call(
        _kpool_body,
        grid=(NBLK,),
        in_specs=[
            pl.BlockSpec((RB, D), lambda i: (i, 0)),
            pl.BlockSpec((1, 1, RB), lambda i: (i, 0, 0)),
            pl.BlockSpec((1, 1, RB), lambda i: (i, 0, 0)),
            pl.BlockSpec((G,), lambda i: (0,)),
        ],
        out_specs=[
            pl.BlockSpec((G,), lambda i: (0,)),
            pl.BlockSpec((G, D), lambda i: (0, 0)),
        ],
        out_shape=[
            jax.ShapeDtypeStruct((G,), jnp.float32),
            jax.ShapeDtypeStruct((G, D), jnp.float32),
        ],
    )(h3, gate, batch, mx)


def _kfinal_body(den_ref, ew_ref, mw1_ref, mb1_ref, mw2_ref, mb2_ref,
                 out_ref):
    pooled = ew_ref[...] / (den_ref[...] + 1e-16)[:, None]
    m1 = jnp.maximum(
        jnp.dot(pooled, mw1_ref[...], preferred_element_type=jnp.float32)
        + mb1_ref[...][None, :], 0.0)
    out_ref[...] = (jnp.dot(m1, mw2_ref[...],
                            preferred_element_type=jnp.float32)
                    + mb2_ref[...][None, :])


def _kfinal(den, ew, mW1, mb1, mW2, mb2):
    return pl.pallas_call(
        _kfinal_body,
        out_shape=jax.ShapeDtypeStruct((G, D), jnp.float32),
    )(den, ew, mW1, mb1, mW2, mb2)


def kernel(x, edge_index, batch, W1, b1, W2, b2, W3, b3, g1, be1, g2, be2,
           gW1, gb1, gW2, gb2, mW1, mb1, mW2, mb2):
    src3 = edge_index[0].reshape(NS, NCH, CHUNK)
    dst3 = edge_index[1].reshape(NS, NCH, CHUNK)
    dst3d = edge_index[1].reshape(NW, NCHUNK, CHUNK)

    deg_parts = _deg_call(dst3d).reshape(NC, NBLK, 1, RB)
    batch3 = batch.reshape(NBLK, 1, RB)
    dinv, hp1 = _k0(deg_parts, x, W1)

    pacc1 = _edge_call(hp1, src3, dst3)
    agg1, st1 = _kep(pacc1, hp1, dinv, b1)
    hp2 = _kmm(agg1, st1, g1, be1, dinv, W2)

    pacc2 = _edge_call(hp2, src3, dst3)
    agg2, st2 = _kep(pacc2, hp2, dinv, b2)
    hp3 = _kmm(agg2, st2, g2, be2, dinv, W3)

    pacc3 = _edge_call(hp3, src3, dst3)
    h3, gate, mx = _kep3(pacc3, hp3, dinv, b3, gW1, gb1, gW2, gb2, batch3)
    den, ew = _kpool(h3, gate, batch3, mx)
    return _kfinal(den, ew, mW1, mb1, mW2, mb2)


# speedup vs baseline: 21.1875x; 1.1805x over previous
import functools

import jax
import jax.numpy as jnp
from jax import lax
from jax.experimental import pallas as pl
from jax.experimental.pallas import tpu as pltpu
from jax.experimental.pallas import tpu_sc as plsc

N = 10000
E = 320000
D = 128
G = 16

NC = 2
NS = 16
NW = NC * NS
EPW = E // NW
CHUNK = 125
NCHUNK = EPW // CHUNK
ROWS_PER_TILE = N // NS

_mesh = plsc.VectorSubcoreMesh(core_axis_name="c", subcore_axis_name="s")


def _deg_body(dst_hbm, out_hbm, dst_v, ones_v, zbuf, acc):
    cid = lax.axis_index("c")
    sid = lax.axis_index("s")
    w = cid * NS + sid

    def zinit(i, _):
        zbuf[pl.ds(i * 16, 16)] = jnp.zeros((16,), jnp.float32)
        return 0

    lax.fori_loop(0, N // 16, zinit, 0)

    @pl.when(sid == 0)
    def _():
        pltpu.sync_copy(zbuf, acc)

    def oinit(i, _):
        ones_v[pl.ds(i * 16, 16)] = jnp.ones((16,), jnp.float32)
        return 0

    lax.fori_loop(0, CHUNK // 16 + 1, oinit, 0)
    plsc.subcore_barrier()

    pltpu.sync_copy(dst_hbm.at[w], dst_v)

    def body(j, _):
        pltpu.sync_copy(ones_v.at[pl.ds(0, CHUNK)], acc.at[dst_v.at[j]],
                        add=True)
        return 0

    lax.fori_loop(0, NCHUNK, body, 0)
    plsc.subcore_barrier()

    @pl.when(sid == 0)
    def _():
        pltpu.sync_copy(acc, out_hbm.at[cid])


_deg_call = pl.kernel(
    _deg_body,
    out_type=jax.ShapeDtypeStruct((NC, N), jnp.float32),
    mesh=_mesh,
    compiler_params=pltpu.CompilerParams(use_tc_tiling_on_sc=False),
    scratch_types=[
        pltpu.VMEM((NCHUNK, CHUNK), jnp.int32),
        pltpu.VMEM((CHUNK + 16 - CHUNK % 16,), jnp.float32),
        pltpu.VMEM((N,), jnp.float32),
        pltpu.VMEM_SHARED((N,), jnp.float32),
    ],
)


CD = D // NC
EPT = E // NS
NCH = EPT // CHUNK


def _edge_body(hp_hbm, src_hbm, dst_hbm, out_hbm,
               src_v, dst_v, buf0, buf1, buf2, acc, sem0, sem1, sem2):
    cid = lax.axis_index("c")
    sid = lax.axis_index("s")

    def zrow(r, _):
        for c in range(CD // 16):
            buf0[r, pl.ds(c * 16, 16)] = jnp.zeros((16,), jnp.float32)
        return 0

    lax.fori_loop(0, CHUNK, zrow, 0)
    base = sid * ROWS_PER_TILE
    for k in range(ROWS_PER_TILE // CHUNK):
        pltpu.sync_copy(buf0, acc.at[pl.ds(base + k * CHUNK, CHUNK)])
    plsc.subcore_barrier()

    pltpu.sync_copy(src_hbm.at[sid], src_v)
    pltpu.sync_copy(dst_hbm.at[sid], dst_v)
    hpc = hp_hbm.at[cid]

    bufs = ((buf0, sem0), (buf1, sem1), (buf2, sem2))
    NB = len(bufs)

    for b, (buf, sem) in enumerate(bufs):
        pltpu.async_copy(hpc.at[src_v.at[b]], buf, sem)

    def body(jj, _):
        for b, (buf, sem) in enumerate(bufs):
            j = jj * NB + b
            pltpu.make_async_copy(hpc.at[src_v.at[j]], buf, sem).wait()
            pltpu.sync_copy(buf, acc.at[dst_v.at[j]], add=True)

            @pl.when(j + NB < NCH)
            def _():
                pltpu.async_copy(hpc.at[src_v.at[j + NB]], buf, sem)

        return 0

    lax.fori_loop(0, NCH // NB, body, 0)

    j = NCH - 1
    pltpu.make_async_copy(hpc.at[src_v.at[j]], buf0, sem0).wait()
    pltpu.sync_copy(buf0, acc.at[dst_v.at[j]], add=True)
    plsc.subcore_barrier()

    for k in range(ROWS_PER_TILE // CHUNK):
        r0 = base + k * CHUNK
        pltpu.sync_copy(acc.at[pl.ds(r0, CHUNK)], buf0)
        pltpu.sync_copy(buf0, out_hbm.at[cid].at[pl.ds(r0, CHUNK)])


_edge_call = pl.kernel(
    _edge_body,
    out_type=jax.ShapeDtypeStruct((NC, N, CD), jnp.float32),
    mesh=_mesh,
    compiler_params=pltpu.CompilerParams(use_tc_tiling_on_sc=False),
    scratch_types=[
        pltpu.VMEM((NCH, CHUNK), jnp.int32),
        pltpu.VMEM((NCH, CHUNK), jnp.int32),
        pltpu.VMEM((CHUNK, CD), jnp.float32),
        pltpu.VMEM((CHUNK, CD), jnp.float32),
        pltpu.VMEM((CHUNK, CD), jnp.float32),
        pltpu.VMEM_SHARED((N, CD), jnp.float32),
        pltpu.SemaphoreType.DMA,
        pltpu.SemaphoreType.DMA,
        pltpu.SemaphoreType.DMA,
    ],
)


RB = 1000
NBLK = N // RB


def _k0_body(degp_ref, x_ref, w1_ref, dinv_ref, hp_ref):
    deg = jnp.sum(degp_ref[:, 0, 0, :], axis=0) + 1.0
    dinv = lax.rsqrt(deg)
    dinv_ref[0, 0, :] = dinv
    h = jnp.dot(x_ref[...], w1_ref[...], preferred_element_type=jnp.float32)
    hp = h * dinv[:, None]
    hp_ref[0] = hp[:, :CD]
    hp_ref[1] = hp[:, CD:]


def _k0(deg_parts, x, W1):
    return pl.pallas_call(
        _k0_body,
        grid=(NBLK,),
        in_specs=[
            pl.BlockSpec((NC, 1, 1, RB), lambda i: (0, i, 0, 0)),
            pl.BlockSpec((RB, D), lambda i: (i, 0)),
            pl.BlockSpec((D, D), lambda i: (0, 0)),
        ],
        out_specs=[
            pl.BlockSpec((1, 1, RB), lambda i: (i, 0, 0)),
            pl.BlockSpec((NC, RB, CD), lambda i: (0, i, 0)),
        ],
        out_shape=[
            jax.ShapeDtypeStruct((NBLK, 1, RB), jnp.float32),
            jax.ShapeDtypeStruct((NC, N, CD), jnp.float32),
        ],
    )(deg_parts, x, W1)


def _kep_body(pacc_ref, hp_ref, dinv_ref, b_ref, agg_ref, st_ref):
    i = pl.program_id(0)
    s = jnp.concatenate([pacc_ref[0] + hp_ref[0], pacc_ref[1] + hp_ref[1]],
                        axis=1)
    agg = s * dinv_ref[0, 0, :][:, None] + b_ref[...][None, :]
    agg_ref[...] = agg
    st = jnp.stack([jnp.sum(agg, axis=0), jnp.sum(agg * agg, axis=0)])

    @pl.when(i == 0)
    def _():
        st_ref[...] = st

    @pl.when(i != 0)
    def _():
        st_ref[...] += st


def _kep(pacc, hp, dinv, b):
    return pl.pallas_call(
        _kep_body,
        grid=(NBLK,),
        in_specs=[
            pl.BlockSpec((NC, RB, CD), lambda i: (0, i, 0)),
            pl.BlockSpec((NC, RB, CD), lambda i: (0, i, 0)),
            pl.BlockSpec((1, 1, RB), lambda i: (i, 0, 0)),
            pl.BlockSpec((D,), lambda i: (0,)),
        ],
        out_specs=[
            pl.BlockSpec((RB, D), lambda i: (i, 0)),
            pl.BlockSpec((2, D), lambda i: (0, 0)),
        ],
        out_shape=[
            jax.ShapeDtypeStruct((N, D), jnp.float32),
            jax.ShapeDtypeStruct((2, D), jnp.float32),
        ],
    )(pacc, hp, dinv, b)


def _kmm_body(agg_ref, st_ref, g_ref, be_ref, dinv_ref, w_ref, hp_ref):
    st = st_ref[...]
    mean = st[0] / N
    var = st[1] / N - mean * mean
    xn = (agg_ref[...] - mean[None, :]) * lax.rsqrt(var + 1e-5)[None, :]
    h = jnp.maximum(xn * g_ref[...][None, :] + be_ref[...][None, :], 0.0)
    hw = jnp.dot(h, w_ref[...], preferred_element_type=jnp.float32)
    hp = hw * dinv_ref[0, 0, :][:, None]
    hp_ref[0] = hp[:, :CD]
    hp_ref[1] = hp[:, CD:]


def _kmm(agg, st, g, be, dinv, W):
    return pl.pallas_call(
        _kmm_body,
        grid=(NBLK,),
        in_specs=[
            pl.BlockSpec((RB, D), lambda i: (i, 0)),
            pl.BlockSpec((2, D), lambda i: (0, 0)),
            pl.BlockSpec((D,), lambda i: (0,)),
            pl.BlockSpec((D,), lambda i: (0,)),
            pl.BlockSpec((1, 1, RB), lambda i: (i, 0, 0)),
            pl.BlockSpec((D, D), lambda i: (0, 0)),
        ],
        out_specs=pl.BlockSpec((NC, RB, CD), lambda i: (0, i, 0)),
        out_shape=jax.ShapeDtypeStruct((NC, N, CD), jnp.float32),
    )(agg, st, g, be, dinv, W)


def _kep3_body(pacc_ref, hp_ref, dinv_ref, b_ref, gw1_ref, gb1_ref,
               gw2_ref, gb2_ref, batch_ref, h3_ref, gate_ref, mx_ref):
    i = pl.program_id(0)
    s = jnp.concatenate([pacc_ref[0] + hp_ref[0], pacc_ref[1] + hp_ref[1]],
                        axis=1)
    h3 = s * dinv_ref[0, 0, :][:, None] + b_ref[...][None, :]
    h3_ref[...] = h3
    gmid = jnp.maximum(
        jnp.dot(h3, gw1_ref[...], preferred_element_type=jnp.float32)
        + gb1_ref[...][None, :], 0.0)
    gate = (jnp.dot(gmid, gw2_ref[...], preferred_element_type=jnp.float32)
            + gb2_ref[...][None, :])[:, 0]
    gate_ref[0, 0, :] = gate
    onehot = batch_ref[0, 0, :][:, None] == lax.broadcasted_iota(
        jnp.int32, (1, G), 1)
    blkmax = jnp.max(jnp.where(onehot, gate[:, None], -jnp.inf), axis=0)

    @pl.when(i == 0)
    def _():
        mx_ref[...] = blkmax

    @pl.when(i != 0)
    def _():
        mx_ref[...] = jnp.maximum(mx_ref[...], blkmax)


def _kep3(pacc, hp, dinv, b, gW1, gb1, gW2, gb2, batch):
    return pl.pallas_call(
        _kep3_body,
        grid=(NBLK,),
        in_specs=[
            pl.BlockSpec((NC, RB, CD), lambda i: (0, i, 0)),
            pl.BlockSpec((NC, RB, CD), lambda i: (0, i, 0)),
            pl.BlockSpec((1, 1, RB), lambda i: (i, 0, 0)),
            pl.BlockSpec((D,), lambda i: (0,)),
            pl.BlockSpec((D, D), lambda i: (0, 0)),
            pl.BlockSpec((D,), lambda i: (0,)),
            pl.BlockSpec((D, 1), lambda i: (0, 0)),
            pl.BlockSpec((1,), lambda i: (0,)),
            pl.BlockSpec((1, 1, RB), lambda i: (i, 0, 0)),
        ],
        out_specs=[
            pl.BlockSpec((RB, D), lambda i: (i, 0)),
            pl.BlockSpec((1, 1, RB), lambda i: (i, 0, 0)),
            pl.BlockSpec((G,), lambda i: (0,)),
        ],
        out_shape=[
            jax.ShapeDtypeStruct((N, D), jnp.float32),
            jax.ShapeDtypeStruct((NBLK, 1, RB), jnp.float32),
            jax.ShapeDtypeStruct((G,), jnp.float32),
        ],
    )(pacc, hp, dinv, b, gW1, gb1, gW2, gb2, batch)


def _kpool_body(h3_ref, gate_ref, batch_ref, mxin_ref, den_ref, ew_ref):
    i = pl.program_id(0)
    mxin = mxin_ref[...]
    mx = jnp.where(jnp.isfinite(mxin), mxin, 0.0)
    onehot = (batch_ref[0, 0, :][:, None] == lax.broadcasted_iota(
        jnp.int32, (1, G), 1)).astype(jnp.float32)
    mxb = jnp.dot(onehot, mx[:, None],
                  preferred_element_type=jnp.float32)[:, 0]
    e = jnp.exp(gate_ref[0, 0, :] - mxb)
    den = jnp.sum(onehot * e[:, None], axis=0)
    ew = lax.dot_general(onehot, h3_ref[...] * e[:, None],
                         (((0,), (0,)), ((), ())),
                         preferred_element_type=jnp.float32)

    @pl.when(i == 0)
    def _():
        den_ref[...] = den
        ew_ref[...] = ew

    @pl.when(i != 0)
    def _():
        den_ref[...] += den
        ew_ref[...] += ew


def _kpool(h3, gate, batch, mx):
    return pl.pallas_call(
        _kpool_body,
        grid=(NBLK,),
        in_specs=[
            pl.BlockSpec((RB, D), lambda i: (i, 0)),
            pl.BlockSpec((1, 1, RB), lambda i: (i, 0, 0)),
            pl.BlockSpec((1, 1, RB), lambda i: (i, 0, 0)),
            pl.BlockSpec((G,), lambda i: (0,)),
        ],
        out_specs=[
            pl.BlockSpec((G,), lambda i: (0,)),
            pl.BlockSpec((G, D), lambda i: (0, 0)),
        ],
        out_shape=[
            jax.ShapeDtypeStruct((G,), jnp.float32),
            jax.ShapeDtypeStruct((G, D), jnp.float32),
        ],
    )(h3, gate, batch, mx)


def _kfinal_body(den_ref, ew_ref, mw1_ref, mb1_ref, mw2_ref, mb2_ref,
                 out_ref):
    pooled = ew_ref[...] / (den_ref[...] + 1e-16)[:, None]
    m1 = jnp.maximum(
        jnp.dot(pooled, mw1_ref[...], preferred_element_type=jnp.float32)
        + mb1_ref[...][None, :], 0.0)
    out_ref[...] = (jnp.dot(m1, mw2_ref[...],
                            preferred_element_type=jnp.float32)
                    + mb2_ref[...][None, :])


def _kfinal(den, ew, mW1, mb1, mW2, mb2):
    return pl.pallas_call(
        _kfinal_body,
        out_shape=jax.ShapeDtypeStruct((G, D), jnp.float32),
    )(den, ew, mW1, mb1, mW2, mb2)


def kernel(x, edge_index, batch, W1, b1, W2, b2, W3, b3, g1, be1, g2, be2,
           gW1, gb1, gW2, gb2, mW1, mb1, mW2, mb2):
    src3 = edge_index[0].reshape(NS, NCH, CHUNK)
    dst3 = edge_index[1].reshape(NS, NCH, CHUNK)
    dst3d = edge_index[1].reshape(NW, NCHUNK, CHUNK)

    deg_parts = _deg_call(dst3d).reshape(NC, NBLK, 1, RB)
    batch3 = batch.reshape(NBLK, 1, RB)
    dinv, hp1 = _k0(deg_parts, x, W1)

    pacc1 = _edge_call(hp1, src3, dst3)
    agg1, st1 = _kep(pacc1, hp1, dinv, b1)
    hp2 = _kmm(agg1, st1, g1, be1, dinv, W2)

    pacc2 = _edge_call(hp2, src3, dst3)
    agg2, st2 = _kep(pacc2, hp2, dinv, b2)
    hp3 = _kmm(agg2, st2, g2, be2, dinv, W3)

    pacc3 = _edge_call(hp3, src3, dst3)
    h3, gate, mx = _kep3(pacc3, hp3, dinv, b3, gW1, gb1, gW2, gb2, batch3)
    den, ew = _kpool(h3, gate, batch3, mx)
    return _kfinal(den, ew, mW1, mb1, mW2, mb2)


# speedup vs baseline: 21.7910x; 1.0285x over previous
import functools

import jax
import jax.numpy as jnp
from jax import lax
from jax.experimental import pallas as pl
from jax.experimental.pallas import tpu as pltpu
from jax.experimental.pallas import tpu_sc as plsc

N = 10000
E = 320000
D = 128
G = 16

NC = 2
NS = 16
NW = NC * NS
EPW = E // NW
CHUNK = 125
NCHUNK = EPW // CHUNK
ROWS_PER_TILE = N // NS

_mesh = plsc.VectorSubcoreMesh(core_axis_name="c", subcore_axis_name="s")


def _deg_body(dst_hbm, out_hbm, dst_v, ones_v, zbuf, acc):
    cid = lax.axis_index("c")
    sid = lax.axis_index("s")
    w = cid * NS + sid

    def zinit(i, _):
        zbuf[pl.ds(i * 16, 16)] = jnp.zeros((16,), jnp.float32)
        return 0

    lax.fori_loop(0, N // 16, zinit, 0)

    @pl.when(sid == 0)
    def _():
        pltpu.sync_copy(zbuf, acc)

    def oinit(i, _):
        ones_v[pl.ds(i * 16, 16)] = jnp.ones((16,), jnp.float32)
        return 0

    lax.fori_loop(0, CHUNK // 16 + 1, oinit, 0)
    plsc.subcore_barrier()

    pltpu.sync_copy(dst_hbm.at[w], dst_v)

    def body(j, _):
        pltpu.sync_copy(ones_v.at[pl.ds(0, CHUNK)], acc.at[dst_v.at[j]],
                        add=True)
        return 0

    lax.fori_loop(0, NCHUNK, body, 0)
    plsc.subcore_barrier()

    @pl.when(sid == 0)
    def _():
        pltpu.sync_copy(acc, out_hbm.at[cid])


_deg_call = pl.kernel(
    _deg_body,
    out_type=jax.ShapeDtypeStruct((NC, N), jnp.float32),
    mesh=_mesh,
    compiler_params=pltpu.CompilerParams(use_tc_tiling_on_sc=False),
    scratch_types=[
        pltpu.VMEM((NCHUNK, CHUNK), jnp.int32),
        pltpu.VMEM((CHUNK + 16 - CHUNK % 16,), jnp.float32),
        pltpu.VMEM((N,), jnp.float32),
        pltpu.VMEM_SHARED((N,), jnp.float32),
    ],
)


CD = D // NC
EPT = E // NS
NCH = EPT // CHUNK


def _edge_body(hp_hbm, src_hbm, dst_hbm, out_hbm,
               src_v, dst_v, buf0, buf1, buf2, acc, sem0, sem1, sem2):
    cid = lax.axis_index("c")
    sid = lax.axis_index("s")

    def zrow(r, _):
        for c in range(CD // 16):
            buf0[r, pl.ds(c * 16, 16)] = jnp.zeros((16,), jnp.float32)
        return 0

    lax.fori_loop(0, CHUNK, zrow, 0)
    base = sid * ROWS_PER_TILE
    for k in range(ROWS_PER_TILE // CHUNK):
        pltpu.sync_copy(buf0, acc.at[pl.ds(base + k * CHUNK, CHUNK)])
    plsc.subcore_barrier()

    pltpu.sync_copy(src_hbm.at[sid], src_v)
    pltpu.sync_copy(dst_hbm.at[sid], dst_v)
    hpc = hp_hbm.at[cid]

    bufs = ((buf0, sem0), (buf1, sem1), (buf2, sem2))
    NB = len(bufs)

    for b, (buf, sem) in enumerate(bufs):
        pltpu.async_copy(hpc.at[src_v.at[b]], buf, sem)

    def body(jj, _):
        for b, (buf, sem) in enumerate(bufs):
            j = jj * NB + b
            pltpu.make_async_copy(hpc.at[src_v.at[j]], buf, sem).wait()
            pltpu.sync_copy(buf, acc.at[dst_v.at[j]], add=True)

            @pl.when(j + NB < NCH)
            def _():
                pltpu.async_copy(hpc.at[src_v.at[j + NB]], buf, sem)

        return 0

    lax.fori_loop(0, NCH // NB, body, 0)

    j = NCH - 1
    pltpu.make_async_copy(hpc.at[src_v.at[j]], buf0, sem0).wait()
    pltpu.sync_copy(buf0, acc.at[dst_v.at[j]], add=True)
    plsc.subcore_barrier()

    for k in range(ROWS_PER_TILE // CHUNK):
        r0 = base + k * CHUNK
        pltpu.sync_copy(acc.at[pl.ds(r0, CHUNK)], buf0)
        pltpu.sync_copy(buf0, out_hbm.at[cid].at[pl.ds(r0, CHUNK)])


_edge_call = pl.kernel(
    _edge_body,
    out_type=jax.ShapeDtypeStruct((NC, N, CD), jnp.float32),
    mesh=_mesh,
    compiler_params=pltpu.CompilerParams(use_tc_tiling_on_sc=False),
    scratch_types=[
        pltpu.VMEM((NCH, CHUNK), jnp.int32),
        pltpu.VMEM((NCH, CHUNK), jnp.int32),
        pltpu.VMEM((CHUNK, CD), jnp.float32),
        pltpu.VMEM((CHUNK, CD), jnp.float32),
        pltpu.VMEM((CHUNK, CD), jnp.float32),
        pltpu.VMEM_SHARED((N, CD), jnp.float32),
        pltpu.SemaphoreType.DMA,
        pltpu.SemaphoreType.DMA,
        pltpu.SemaphoreType.DMA,
    ],
)


RB = 1000
NBLK = N // RB


def _k0_body(degp_ref, x_ref, w1_ref, dinv_ref, hp_ref):
    deg = jnp.sum(degp_ref[:, 0, 0, :], axis=0) + 1.0
    dinv = lax.rsqrt(deg)
    dinv_ref[0, 0, :] = dinv
    h = jnp.dot(x_ref[...], w1_ref[...], preferred_element_type=jnp.float32)
    hp = h * dinv[:, None]
    hp_ref[0] = hp[:, :CD]
    hp_ref[1] = hp[:, CD:]


def _k0(deg_parts, x, W1):
    return pl.pallas_call(
        _k0_body,
        grid=(NBLK,),
        in_specs=[
            pl.BlockSpec((NC, 1, 1, RB), lambda i: (0, i, 0, 0)),
            pl.BlockSpec((RB, D), lambda i: (i, 0)),
            pl.BlockSpec((D, D), lambda i: (0, 0)),
        ],
        out_specs=[
            pl.BlockSpec((1, 1, RB), lambda i: (i, 0, 0)),
            pl.BlockSpec((NC, RB, CD), lambda i: (0, i, 0)),
        ],
        out_shape=[
            jax.ShapeDtypeStruct((NBLK, 1, RB), jnp.float32),
            jax.ShapeDtypeStruct((NC, N, CD), jnp.float32),
        ],
    )(deg_parts, x, W1)


def _fuse_mid_body(pacc_ref, hp_ref, dinv_ref, b_ref, g_ref, be_ref,
                   w_ref, out_ref, agg_s, st_s):
    i = pl.program_id(0)

    @pl.when(i < NBLK)
    def _():
        s = jnp.concatenate(
            [pacc_ref[0] + hp_ref[0], pacc_ref[1] + hp_ref[1]], axis=1)
        agg = s * dinv_ref[0, 0, :][:, None] + b_ref[...][None, :]
        agg_s[pl.ds(i, 1)] = agg[None]
        st = jnp.stack([jnp.sum(agg, axis=0), jnp.sum(agg * agg, axis=0)])

        @pl.when(i == 0)
        def _():
            st_s[...] = st

        @pl.when(i != 0)
        def _():
            st_s[...] += st

    @pl.when(i >= NBLK)
    def _():
        k = i - NBLK
        st = st_s[...]
        mean = st[0] / N
        var = st[1] / N - mean * mean
        agg = agg_s[pl.ds(k, 1)][0]
        xn = (agg - mean[None, :]) * lax.rsqrt(var + 1e-5)[None, :]
        h = jnp.maximum(xn * g_ref[...][None, :] + be_ref[...][None, :], 0.0)
        hw = jnp.dot(h, w_ref[...], preferred_element_type=jnp.float32)
        hp = hw * dinv_ref[0, 0, :][:, None]
        out_ref[0] = hp[:, :CD]
        out_ref[1] = hp[:, CD:]


def _fuse_mid(pacc, hp, dinv, b, g, be, W):
    blk = lambda i: jnp.where(i < NBLK, i, i - NBLK)
    blka = lambda i: jnp.minimum(i, NBLK - 1)
    return pl.pallas_call(
        _fuse_mid_body,
        grid=(2 * NBLK,),
        in_specs=[
            pl.BlockSpec((NC, RB, CD), lambda i: (0, blka(i), 0)),
            pl.BlockSpec((NC, RB, CD), lambda i: (0, blka(i), 0)),
            pl.BlockSpec((1, 1, RB), lambda i: (blk(i), 0, 0)),
            pl.BlockSpec((D,), lambda i: (0,)),
            pl.BlockSpec((D,), lambda i: (0,)),
            pl.BlockSpec((D,), lambda i: (0,)),
            pl.BlockSpec((D, D), lambda i: (0, 0)),
        ],
        out_specs=pl.BlockSpec((NC, RB, CD),
                               lambda i: (0, jnp.maximum(i - NBLK, 0), 0)),
        out_shape=jax.ShapeDtypeStruct((NC, N, CD), jnp.float32),
        scratch_shapes=[
            pltpu.VMEM((NBLK, RB, D), jnp.float32),
            pltpu.VMEM((2, D), jnp.float32),
        ],
    )(pacc, hp, dinv, b, g, be, W)


def _fuse_tail_body(pacc_ref, hp_ref, dinv_ref, b_ref, gw1_ref, gb1_ref,
                    gw2_ref, gb2_ref, batch_ref, mw1_ref, mb1_ref, mw2_ref,
                    mb2_ref, out_ref, h3_s, gate_s, mx_s, den_s, ew_s):
    i = pl.program_id(0)

    @pl.when(i < NBLK)
    def _():
        s = jnp.concatenate(
            [pacc_ref[0] + hp_ref[0], pacc_ref[1] + hp_ref[1]], axis=1)
        h3 = s * dinv_ref[0, 0, :][:, None] + b_ref[...][None, :]
        h3_s[pl.ds(i, 1)] = h3[None]
        gmid = jnp.maximum(
            jnp.dot(h3, gw1_ref[...], preferred_element_type=jnp.float32)
            + gb1_ref[...][None, :], 0.0)
        gate = (jnp.dot(gmid, gw2_ref[...],
                        preferred_element_type=jnp.float32)
                + gb2_ref[...][None, :])[:, 0]
        gate_s[pl.ds(i, 1)] = gate[None]
        onehot = batch_ref[0, 0, :][:, None] == lax.broadcasted_iota(
            jnp.int32, (1, G), 1)
        blkmax = jnp.max(jnp.where(onehot, gate[:, None], -jnp.inf),
                         axis=0, keepdims=True)

        @pl.when(i == 0)
        def _():
            mx_s[...] = blkmax

        @pl.when(i != 0)
        def _():
            mx_s[...] = jnp.maximum(mx_s[...], blkmax)

    @pl.when(jnp.logical_and(i >= NBLK, i < 2 * NBLK))
    def _():
        k = i - NBLK
        mxin = mx_s[0]
        mx = jnp.where(jnp.isfinite(mxin), mxin, 0.0)
        onehot = (batch_ref[0, 0, :][:, None] == lax.broadcasted_iota(
            jnp.int32, (1, G), 1)).astype(jnp.float32)
        mxb = jnp.dot(onehot, mx[:, None],
                      preferred_element_type=jnp.float32)[:, 0]
        e = jnp.exp(gate_s[pl.ds(k, 1)][0] - mxb)
        den = jnp.sum(onehot * e[:, None], axis=0, keepdims=True)
        ew = lax.dot_general(onehot, h3_s[pl.ds(k, 1)][0] * e[:, None],
                             (((0,), (0,)), ((), ())),
                             preferred_element_type=jnp.float32)

        @pl.when(k == 0)
        def _():
            den_s[...] = den
            ew_s[...] = ew

        @pl.when(k != 0)
        def _():
            den_s[...] += den
            ew_s[...] += ew

    @pl.when(i == 2 * NBLK)
    def _():
        pooled = ew_s[...] / (den_s[0] + 1e-16)[:, None]
        m1 = jnp.maximum(
            jnp.dot(pooled, mw1_ref[...], preferred_element_type=jnp.float32)
            + mb1_ref[...][None, :], 0.0)
        out_ref[...] = (jnp.dot(m1, mw2_ref[...],
                                preferred_element_type=jnp.float32)
                        + mb2_ref[...][None, :])


def _fuse_tail(pacc, hp, dinv, b, gW1, gb1, gW2, gb2, batch3,
               mW1, mb1, mW2, mb2):
    blk = lambda i: jnp.where(i < NBLK, i,
                              jnp.where(i < 2 * NBLK, i - NBLK, 0))
    blka = lambda i: jnp.minimum(i, NBLK - 1)
    return pl.pallas_call(
        _fuse_tail_body,
        grid=(2 * NBLK + 1,),
        in_specs=[
            pl.BlockSpec((NC, RB, CD), lambda i: (0, blka(i), 0)),
            pl.BlockSpec((NC, RB, CD), lambda i: (0, blka(i), 0)),
            pl.BlockSpec((1, 1, RB), lambda i: (blk(i), 0, 0)),
            pl.BlockSpec((D,), lambda i: (0,)),
            pl.BlockSpec((D, D), lambda i: (0, 0)),
            pl.BlockSpec((D,), lambda i: (0,)),
            pl.BlockSpec((D, 1), lambda i: (0, 0)),
            pl.BlockSpec((1,), lambda i: (0,)),
            pl.BlockSpec((1, 1, RB), lambda i: (blk(i), 0, 0)),
            pl.BlockSpec((D, D), lambda i: (0, 0)),
            pl.BlockSpec((D,), lambda i: (0,)),
            pl.BlockSpec((D, D), lambda i: (0, 0)),
            pl.BlockSpec((D,), lambda i: (0,)),
        ],
        out_specs=pl.BlockSpec((G, D), lambda i: (0, 0)),
        out_shape=jax.ShapeDtypeStruct((G, D), jnp.float32),
        scratch_shapes=[
            pltpu.VMEM((NBLK, RB, D), jnp.float32),
            pltpu.VMEM((NBLK, RB), jnp.float32),
            pltpu.VMEM((1, G), jnp.float32),
            pltpu.VMEM((1, G), jnp.float32),
            pltpu.VMEM((G, D), jnp.float32),
        ],
    )(pacc, hp, dinv, b, gW1, gb1, gW2, gb2, batch3, mW1, mb1, mW2, mb2)


def kernel(x, edge_index, batch, W1, b1, W2, b2, W3, b3, g1, be1, g2, be2,
           gW1, gb1, gW2, gb2, mW1, mb1, mW2, mb2):
    src3 = edge_index[0].reshape(NS, NCH, CHUNK)
    dst3 = edge_index[1].reshape(NS, NCH, CHUNK)
    dst3d = edge_index[1].reshape(NW, NCHUNK, CHUNK)

    deg_parts = _deg_call(dst3d).reshape(NC, NBLK, 1, RB)
    batch3 = batch.reshape(NBLK, 1, RB)
    dinv, hp1 = _k0(deg_parts, x, W1)

    pacc1 = _edge_call(hp1, src3, dst3)
    hp2 = _fuse_mid(pacc1, hp1, dinv, b1, g1, be1, W2)

    pacc2 = _edge_call(hp2, src3, dst3)
    hp3 = _fuse_mid(pacc2, hp2, dinv, b2, g2, be2, W3)

    pacc3 = _edge_call(hp3, src3, dst3)
    return _fuse_tail(pacc3, hp3, dinv, b3, gW1, gb1, gW2, gb2, batch3,
                      mW1, mb1, mW2, mb2)


# speedup vs baseline: 22.8564x; 1.0489x over previous
import functools

import jax
import jax.numpy as jnp
from jax import lax
from jax.experimental import pallas as pl
from jax.experimental.pallas import tpu as pltpu
from jax.experimental.pallas import tpu_sc as plsc

N = 10000
E = 320000
D = 128
G = 16

NC = 2
NS = 16
NW = NC * NS
EPW = E // NW
CHUNK = 125
NCHUNK = EPW // CHUNK
ROWS_PER_TILE = N // NS

_mesh = plsc.VectorSubcoreMesh(core_axis_name="c", subcore_axis_name="s")


def _deg_body(dst_hbm, out_hbm, dst_v, ones_v, zbuf, acc):
    cid = lax.axis_index("c")
    sid = lax.axis_index("s")
    w = cid * NS + sid

    def zinit(i, _):
        zbuf[pl.ds(i * 16, 16)] = jnp.zeros((16,), jnp.float32)
        return 0

    lax.fori_loop(0, N // 16, zinit, 0)

    @pl.when(sid == 0)
    def _():
        pltpu.sync_copy(zbuf, acc)

    def oinit(i, _):
        ones_v[pl.ds(i * 16, 16)] = jnp.ones((16,), jnp.float32)
        return 0

    lax.fori_loop(0, CHUNK // 16 + 1, oinit, 0)
    plsc.subcore_barrier()

    pltpu.sync_copy(dst_hbm.at[w], dst_v)

    def body(j, _):
        pltpu.sync_copy(ones_v.at[pl.ds(0, CHUNK)], acc.at[dst_v.at[j]],
                        add=True)
        return 0

    lax.fori_loop(0, NCHUNK, body, 0)
    plsc.subcore_barrier()

    @pl.when(sid == 0)
    def _():
        pltpu.sync_copy(acc, out_hbm.at[cid])


_deg_call = pl.kernel(
    _deg_body,
    out_type=jax.ShapeDtypeStruct((NC, N), jnp.float32),
    mesh=_mesh,
    compiler_params=pltpu.CompilerParams(use_tc_tiling_on_sc=False),
    scratch_types=[
        pltpu.VMEM((NCHUNK, CHUNK), jnp.int32),
        pltpu.VMEM((CHUNK + 16 - CHUNK % 16,), jnp.float32),
        pltpu.VMEM((N,), jnp.float32),
        pltpu.VMEM_SHARED((N,), jnp.float32),
    ],
)


CD = D // NC
EPT = E // NS
NCH = EPT // CHUNK


def _edge_body(hp_hbm, src_hbm, dst_hbm, out_hbm,
               src_v, dst_v, buf0, buf1, buf2, acc, sem0, sem1, sem2):
    cid = lax.axis_index("c")
    sid = lax.axis_index("s")

    def zrow(r, _):
        for c in range(CD // 16):
            buf0[r, pl.ds(c * 16, 16)] = jnp.zeros((16,), jnp.float32)
        return 0

    lax.fori_loop(0, CHUNK, zrow, 0)
    base = sid * ROWS_PER_TILE
    for k in range(ROWS_PER_TILE // CHUNK):
        pltpu.sync_copy(buf0, acc.at[pl.ds(base + k * CHUNK, CHUNK)])
    plsc.subcore_barrier()

    pltpu.sync_copy(src_hbm.at[sid], src_v)
    pltpu.sync_copy(dst_hbm.at[sid], dst_v)
    hpc = hp_hbm.at[cid]

    bufs = ((buf0, sem0), (buf1, sem1), (buf2, sem2))
    NB = len(bufs)

    for b, (buf, sem) in enumerate(bufs):
        pltpu.async_copy(hpc.at[src_v.at[b]], buf, sem)

    def body(jj, _):
        for b, (buf, sem) in enumerate(bufs):
            j = jj * NB + b
            pltpu.make_async_copy(hpc.at[src_v.at[j]], buf, sem).wait()
            pltpu.sync_copy(buf, acc.at[dst_v.at[j]], add=True)

            @pl.when(j + NB < NCH)
            def _():
                pltpu.async_copy(hpc.at[src_v.at[j + NB]], buf, sem)

        return 0

    lax.fori_loop(0, NCH // NB, body, 0)

    j = NCH - 1
    pltpu.make_async_copy(hpc.at[src_v.at[j]], buf0, sem0).wait()
    pltpu.sync_copy(buf0, acc.at[dst_v.at[j]], add=True)
    plsc.subcore_barrier()

    for k in range(ROWS_PER_TILE // CHUNK):
        r0 = base + k * CHUNK
        pltpu.sync_copy(acc.at[pl.ds(r0, CHUNK)], buf0)
        pltpu.sync_copy(buf0, out_hbm.at[cid].at[pl.ds(r0, CHUNK)])


_edge_call = pl.kernel(
    _edge_body,
    out_type=jax.ShapeDtypeStruct((NC, N, CD), jnp.float32),
    mesh=_mesh,
    compiler_params=pltpu.CompilerParams(use_tc_tiling_on_sc=False),
    scratch_types=[
        pltpu.VMEM((NCH, CHUNK), jnp.int32),
        pltpu.VMEM((NCH, CHUNK), jnp.int32),
        pltpu.VMEM((CHUNK, CD), jnp.float32),
        pltpu.VMEM((CHUNK, CD), jnp.float32),
        pltpu.VMEM((CHUNK, CD), jnp.float32),
        pltpu.VMEM_SHARED((N, CD), jnp.float32),
        pltpu.SemaphoreType.DMA,
        pltpu.SemaphoreType.DMA,
        pltpu.SemaphoreType.DMA,
    ],
)


def _k0_body(degp_ref, x_ref, w1_ref, dinv_ref, hp_ref):
    deg = jnp.sum(degp_ref[...], axis=0) + 1.0
    dinv = lax.rsqrt(deg)
    dinv_ref[...] = dinv
    h = jnp.dot(x_ref[...], w1_ref[...], preferred_element_type=jnp.float32)
    hp = h * dinv[:, None]
    hp_ref[0] = hp[:, :CD]
    hp_ref[1] = hp[:, CD:]


def _k0(deg_parts, x, W1):
    return pl.pallas_call(
        _k0_body,
        out_shape=[
            jax.ShapeDtypeStruct((N,), jnp.float32),
            jax.ShapeDtypeStruct((NC, N, CD), jnp.float32),
        ],
    )(deg_parts, x, W1)


def _fuse_mid_body(pacc_ref, hp_ref, dinv_ref, b_ref, g_ref, be_ref,
                   w_ref, out_ref):
    dinv = dinv_ref[...]
    s = jnp.concatenate([pacc_ref[0] + hp_ref[0], pacc_ref[1] + hp_ref[1]],
                        axis=1)
    agg = s * dinv[:, None] + b_ref[...][None, :]
    mean = jnp.mean(agg, axis=0)
    var = jnp.mean(agg * agg, axis=0) - mean * mean
    xn = (agg - mean[None, :]) * lax.rsqrt(var + 1e-5)[None, :]
    h = jnp.maximum(xn * g_ref[...][None, :] + be_ref[...][None, :], 0.0)
    hw = jnp.dot(h, w_ref[...], preferred_element_type=jnp.float32)
    hp = hw * dinv[:, None]
    out_ref[0] = hp[:, :CD]
    out_ref[1] = hp[:, CD:]


def _fuse_mid(pacc, hp, dinv, b, g, be, W):
    return pl.pallas_call(
        _fuse_mid_body,
        out_shape=jax.ShapeDtypeStruct((NC, N, CD), jnp.float32),
    )(pacc, hp, dinv, b, g, be, W)


def _fuse_tail_body(pacc_ref, hp_ref, dinv_ref, b_ref, gw1_ref, gb1_ref,
                    gw2_ref, gb2_ref, batch_ref, mw1_ref, mb1_ref, mw2_ref,
                    mb2_ref, out_ref):
    s = jnp.concatenate([pacc_ref[0] + hp_ref[0], pacc_ref[1] + hp_ref[1]],
                        axis=1)
    h3 = s * dinv_ref[...][:, None] + b_ref[...][None, :]
    gmid = jnp.maximum(
        jnp.dot(h3, gw1_ref[...], preferred_element_type=jnp.float32)
        + gb1_ref[...][None, :], 0.0)
    gate = (jnp.dot(gmid, gw2_ref[...], preferred_element_type=jnp.float32)
            + gb2_ref[...][None, :])[:, 0]
    oh = batch_ref[...][:, None] == lax.broadcasted_iota(jnp.int32, (1, G), 1)
    mx = jnp.max(jnp.where(oh, gate[:, None], -jnp.inf), axis=0)
    mx = jnp.where(jnp.isfinite(mx), mx, 0.0)
    ohf = oh.astype(jnp.float32)
    mxb = jnp.dot(ohf, mx[:, None], preferred_element_type=jnp.float32)[:, 0]
    e = jnp.exp(gate - mxb)
    den = jnp.sum(ohf * e[:, None], axis=0)
    ew = lax.dot_general(ohf, h3 * e[:, None], (((0,), (0,)), ((), ())),
                         preferred_element_type=jnp.float32)
    pooled = ew / (den + 1e-16)[:, None]
    m1 = jnp.maximum(
        jnp.dot(pooled, mw1_ref[...], preferred_element_type=jnp.float32)
        + mb1_ref[...][None, :], 0.0)
    out_ref[...] = (jnp.dot(m1, mw2_ref[...],
                            preferred_element_type=jnp.float32)
                    + mb2_ref[...][None, :])


def _fuse_tail(pacc, hp, dinv, b, gW1, gb1, gW2, gb2, batch,
               mW1, mb1, mW2, mb2):
    return pl.pallas_call(
        _fuse_tail_body,
        out_shape=jax.ShapeDtypeStruct((G, D), jnp.float32),
    )(pacc, hp, dinv, b, gW1, gb1, gW2, gb2, batch, mW1, mb1, mW2, mb2)


def kernel(x, edge_index, batch, W1, b1, W2, b2, W3, b3, g1, be1, g2, be2,
           gW1, gb1, gW2, gb2, mW1, mb1, mW2, mb2):
    src3 = edge_index[0].reshape(NS, NCH, CHUNK)
    dst3 = edge_index[1].reshape(NS, NCH, CHUNK)
    dst3d = edge_index[1].reshape(NW, NCHUNK, CHUNK)

    deg_parts = _deg_call(dst3d)
    dinv, hp1 = _k0(deg_parts, x, W1)

    pacc1 = _edge_call(hp1, src3, dst3)
    hp2 = _fuse_mid(pacc1, hp1, dinv, b1, g1, be1, W2)

    pacc2 = _edge_call(hp2, src3, dst3)
    hp3 = _fuse_mid(pacc2, hp2, dinv, b2, g2, be2, W3)

    pacc3 = _edge_call(hp3, src3, dst3)
    return _fuse_tail(pacc3, hp3, dinv, b3, gW1, gb1, gW2, gb2, batch,
                      mW1, mb1, mW2, mb2)


# speedup vs baseline: 24.2455x; 1.0608x over previous
import functools

import jax
import jax.numpy as jnp
from jax import lax
from jax.experimental import pallas as pl
from jax.experimental.pallas import tpu as pltpu
from jax.experimental.pallas import tpu_sc as plsc

N = 10000
E = 320000
D = 128
G = 16

NC = 2
NS = 16
NW = NC * NS
EPW = E // NW
CHUNK = 125
NCHUNK = EPW // CHUNK
ROWS_PER_TILE = N // NS

_mesh = plsc.VectorSubcoreMesh(core_axis_name="c", subcore_axis_name="s")


def _deg_body(dst_hbm, out_hbm, dst_v, ones_v, zbuf, acc):
    cid = lax.axis_index("c")
    sid = lax.axis_index("s")
    w = cid * NS + sid

    def zinit(i, _):
        zbuf[pl.ds(i * 16, 16)] = jnp.zeros((16,), jnp.float32)
        return 0

    lax.fori_loop(0, N // 16, zinit, 0)

    @pl.when(sid == 0)
    def _():
        pltpu.sync_copy(zbuf, acc)

    def oinit(i, _):
        ones_v[pl.ds(i * 16, 16)] = jnp.ones((16,), jnp.float32)
        return 0

    lax.fori_loop(0, CHUNK // 16 + 1, oinit, 0)
    plsc.subcore_barrier()

    pltpu.sync_copy(dst_hbm.at[w], dst_v)

    def body(j, _):
        pltpu.sync_copy(ones_v.at[pl.ds(0, CHUNK)], acc.at[dst_v.at[j]],
                        add=True)
        return 0

    lax.fori_loop(0, NCHUNK, body, 0)
    plsc.subcore_barrier()

    @pl.when(sid == 0)
    def _():
        pltpu.sync_copy(acc, out_hbm.at[cid])


_deg_call = pl.kernel(
    _deg_body,
    out_type=jax.ShapeDtypeStruct((NC, N), jnp.float32),
    mesh=_mesh,
    compiler_params=pltpu.CompilerParams(use_tc_tiling_on_sc=False),
    scratch_types=[
        pltpu.VMEM((NCHUNK, CHUNK), jnp.int32),
        pltpu.VMEM((CHUNK + 16 - CHUNK % 16,), jnp.float32),
        pltpu.VMEM((N,), jnp.float32),
        pltpu.VMEM_SHARED((N,), jnp.float32),
    ],
)


CD = D // NC
EPW2 = E // NW
NCH2 = EPW2 // CHUNK
HSTG = NCH2 // 2


def _edge_body(hp_hbm, src_hbm, dst_hbm, out_hbm,
               src_v, dst_v, buf0, buf1, acc, sem0, sem1):
    cid = lax.axis_index("c")
    sid = lax.axis_index("s")
    w = cid * NS + sid

    def zrow(r, _):
        for c in range(D // 16):
            buf0[r, pl.ds(c * 16, 16)] = jnp.zeros((16,), jnp.float32)
        return 0

    lax.fori_loop(0, CHUNK, zrow, 0)
    base = sid * ROWS_PER_TILE
    for k in range(ROWS_PER_TILE // CHUNK):
        pltpu.sync_copy(buf0, acc.at[pl.ds(base + k * CHUNK, CHUNK)])
    plsc.subcore_barrier()

    bufs = ((buf0, sem0), (buf1, sem1))
    NB = len(bufs)

    for half in range(2):
        pltpu.sync_copy(src_hbm.at[w].at[pl.ds(half * HSTG, HSTG)], src_v)
        pltpu.sync_copy(dst_hbm.at[w].at[pl.ds(half * HSTG, HSTG)], dst_v)

        for b, (buf, sem) in enumerate(bufs):
            pltpu.async_copy(hp_hbm.at[src_v.at[b]], buf, sem)

        def body(jj, _):
            for b, (buf, sem) in enumerate(bufs):
                j = jj * NB + b
                pltpu.make_async_copy(hp_hbm.at[src_v.at[j]], buf,
                                      sem).wait()
                pltpu.sync_copy(buf, acc.at[dst_v.at[j]], add=True)

                @pl.when(j + NB < HSTG)
                def _():
                    pltpu.async_copy(hp_hbm.at[src_v.at[j + NB]], buf, sem)

            return 0

        lax.fori_loop(0, HSTG // NB, body, 0)

    plsc.subcore_barrier()

    for k in range(ROWS_PER_TILE // CHUNK):
        r0 = base + k * CHUNK
        pltpu.sync_copy(acc.at[pl.ds(r0, CHUNK)], buf0)
        pltpu.sync_copy(buf0, out_hbm.at[cid].at[pl.ds(r0, CHUNK)])


_edge_call = pl.kernel(
    _edge_body,
    out_type=jax.ShapeDtypeStruct((NC, N, D), jnp.float32),
    mesh=_mesh,
    compiler_params=pltpu.CompilerParams(use_tc_tiling_on_sc=False),
    scratch_types=[
        pltpu.VMEM((HSTG, CHUNK), jnp.int32),
        pltpu.VMEM((HSTG, CHUNK), jnp.int32),
        pltpu.VMEM((CHUNK, D), jnp.float32),
        pltpu.VMEM((CHUNK, D), jnp.float32),
        pltpu.VMEM_SHARED((N, D), jnp.float32),
        pltpu.SemaphoreType.DMA,
        pltpu.SemaphoreType.DMA,
    ],
)


def _k0_body(degp_ref, x_ref, w1_ref, dinv_ref, hp_ref):
    deg = jnp.sum(degp_ref[...], axis=0) + 1.0
    dinv = lax.rsqrt(deg)
    dinv_ref[...] = dinv
    h = jnp.dot(x_ref[...], w1_ref[...], preferred_element_type=jnp.float32)
    hp_ref[...] = h * dinv[:, None]


def _k0(deg_parts, x, W1):
    return pl.pallas_call(
        _k0_body,
        out_shape=[
            jax.ShapeDtypeStruct((N,), jnp.float32),
            jax.ShapeDtypeStruct((N, D), jnp.float32),
        ],
    )(deg_parts, x, W1)


def _fuse_mid_body(pacc_ref, hp_ref, dinv_ref, b_ref, g_ref, be_ref,
                   w_ref, out_ref):
    dinv = dinv_ref[...]
    s = pacc_ref[0] + pacc_ref[1] + hp_ref[...]
    agg = s * dinv[:, None] + b_ref[...][None, :]
    mean = jnp.mean(agg, axis=0)
    var = jnp.mean(agg * agg, axis=0) - mean * mean
    xn = (agg - mean[None, :]) * lax.rsqrt(var + 1e-5)[None, :]
    h = jnp.maximum(xn * g_ref[...][None, :] + be_ref[...][None, :], 0.0)
    hw = jnp.dot(h, w_ref[...], preferred_element_type=jnp.float32)
    out_ref[...] = hw * dinv[:, None]


def _fuse_mid(pacc, hp, dinv, b, g, be, W):
    return pl.pallas_call(
        _fuse_mid_body,
        out_shape=jax.ShapeDtypeStruct((N, D), jnp.float32),
    )(pacc, hp, dinv, b, g, be, W)


def _fuse_tail_body(pacc_ref, hp_ref, dinv_ref, b_ref, gw1_ref, gb1_ref,
                    gw2_ref, gb2_ref, batch_ref, mw1_ref, mb1_ref, mw2_ref,
                    mb2_ref, out_ref):
    s = pacc_ref[0] + pacc_ref[1] + hp_ref[...]
    h3 = s * dinv_ref[...][:, None] + b_ref[...][None, :]
    gmid = jnp.maximum(
        jnp.dot(h3, gw1_ref[...], preferred_element_type=jnp.float32)
        + gb1_ref[...][None, :], 0.0)
    gate = (jnp.dot(gmid, gw2_ref[...], preferred_element_type=jnp.float32)
            + gb2_ref[...][None, :])[:, 0]
    oh = batch_ref[...][:, None] == lax.broadcasted_iota(jnp.int32, (1, G), 1)
    mx = jnp.max(jnp.where(oh, gate[:, None], -jnp.inf), axis=0)
    mx = jnp.where(jnp.isfinite(mx), mx, 0.0)
    ohf = oh.astype(jnp.float32)
    mxb = jnp.dot(ohf, mx[:, None], preferred_element_type=jnp.float32)[:, 0]
    e = jnp.exp(gate - mxb)
    den = jnp.sum(ohf * e[:, None], axis=0)
    ew = lax.dot_general(ohf, h3 * e[:, None], (((0,), (0,)), ((), ())),
                         preferred_element_type=jnp.float32)
    pooled = ew / (den + 1e-16)[:, None]
    m1 = jnp.maximum(
        jnp.dot(pooled, mw1_ref[...], preferred_element_type=jnp.float32)
        + mb1_ref[...][None, :], 0.0)
    out_ref[...] = (jnp.dot(m1, mw2_ref[...],
                            preferred_element_type=jnp.float32)
                    + mb2_ref[...][None, :])


def _fuse_tail(pacc, hp, dinv, b, gW1, gb1, gW2, gb2, batch,
               mW1, mb1, mW2, mb2):
    return pl.pallas_call(
        _fuse_tail_body,
        out_shape=jax.ShapeDtypeStruct((G, D), jnp.float32),
    )(pacc, hp, dinv, b, gW1, gb1, gW2, gb2, batch, mW1, mb1, mW2, mb2)


def kernel(x, edge_index, batch, W1, b1, W2, b2, W3, b3, g1, be1, g2, be2,
           gW1, gb1, gW2, gb2, mW1, mb1, mW2, mb2):
    src3 = edge_index[0].reshape(NW, NCH2, CHUNK)
    dst3 = edge_index[1].reshape(NW, NCH2, CHUNK)
    dst3d = dst3

    deg_parts = _deg_call(dst3d)
    dinv, hp1 = _k0(deg_parts, x, W1)

    pacc1 = _edge_call(hp1, src3, dst3)
    hp2 = _fuse_mid(pacc1, hp1, dinv, b1, g1, be1, W2)

    pacc2 = _edge_call(hp2, src3, dst3)
    hp3 = _fuse_mid(pacc2, hp2, dinv, b2, g2, be2, W3)

    pacc3 = _edge_call(hp3, src3, dst3)
    return _fuse_tail(pacc3, hp3, dinv, b3, gW1, gb1, gW2, gb2, batch,
                      mW1, mb1, mW2, mb2)


# speedup vs baseline: 24.9352x; 1.0284x over previous
import functools

import jax
import jax.numpy as jnp
from jax import lax
from jax.experimental import pallas as pl
from jax.experimental.pallas import tpu as pltpu
from jax.experimental.pallas import tpu_sc as plsc

N = 10000
E = 320000
D = 128
G = 16

NC = 2
NS = 16
NW = NC * NS
EPW = E // NW
CHUNK = 125
NCHUNK = EPW // CHUNK
ROWS_PER_TILE = N // NS

_mesh = plsc.VectorSubcoreMesh(core_axis_name="c", subcore_axis_name="s")


def _deg_body(dst_hbm, out_hbm, dst_v, ones_v, zbuf, acc):
    cid = lax.axis_index("c")
    sid = lax.axis_index("s")
    w = cid * NS + sid

    def zinit(i, _):
        zbuf[pl.ds(i * 16, 16)] = jnp.zeros((16,), jnp.float32)
        return 0

    lax.fori_loop(0, N // 16, zinit, 0)

    @pl.when(sid == 0)
    def _():
        pltpu.sync_copy(zbuf, acc)

    def oinit(i, _):
        ones_v[pl.ds(i * 16, 16)] = jnp.ones((16,), jnp.float32)
        return 0

    lax.fori_loop(0, CHUNK // 16 + 1, oinit, 0)
    plsc.subcore_barrier()

    pltpu.sync_copy(dst_hbm.at[w], dst_v)

    def body(j, _):
        pltpu.sync_copy(ones_v.at[pl.ds(0, CHUNK)], acc.at[dst_v.at[j]],
                        add=True)
        return 0

    lax.fori_loop(0, NCHUNK, body, 0)
    plsc.subcore_barrier()

    @pl.when(sid == 0)
    def _():
        pltpu.sync_copy(acc, out_hbm.at[cid])


_deg_call = pl.kernel(
    _deg_body,
    out_type=jax.ShapeDtypeStruct((NC, N), jnp.float32),
    mesh=_mesh,
    compiler_params=pltpu.CompilerParams(use_tc_tiling_on_sc=False),
    scratch_types=[
        pltpu.VMEM((NCHUNK, CHUNK), jnp.int32),
        pltpu.VMEM((CHUNK + 16 - CHUNK % 16,), jnp.float32),
        pltpu.VMEM((N,), jnp.float32),
        pltpu.VMEM_SHARED((N,), jnp.float32),
    ],
)


CD = D // NC
EPW2 = E // NW
ECH = 100
NCH2 = EPW2 // ECH
NSTG = 4
HSTG = NCH2 // NSTG


def _edge_body(hp_hbm, src_hbm, dst_hbm, out_hbm,
               src_v, dst_v, buf0, buf1, buf2, acc, sem0, sem1, sem2):
    cid = lax.axis_index("c")
    sid = lax.axis_index("s")
    w = cid * NS + sid

    def zrow(r, _):
        for c in range(D // 16):
            buf0[r, pl.ds(c * 16, 16)] = jnp.zeros((16,), jnp.float32)
        return 0

    lax.fori_loop(0, ECH, zrow, 0)
    base = sid * ROWS_PER_TILE
    for k in range(6):
        pltpu.sync_copy(buf0, acc.at[pl.ds(base + k * ECH, ECH)])
    pltpu.sync_copy(buf0.at[pl.ds(0, 25)], acc.at[pl.ds(base + 600, 25)])
    plsc.subcore_barrier()

    bufs = ((buf0, sem0), (buf1, sem1), (buf2, sem2))
    NB = len(bufs)

    for q in range(NSTG):
        pltpu.sync_copy(src_hbm.at[w].at[pl.ds(q * HSTG, HSTG)], src_v)
        pltpu.sync_copy(dst_hbm.at[w].at[pl.ds(q * HSTG, HSTG)], dst_v)

        for b, (buf, sem) in enumerate(bufs):
            pltpu.async_copy(hp_hbm.at[src_v.at[b]], buf, sem)

        def body(jj, _):
            for b, (buf, sem) in enumerate(bufs):
                j = jj * NB + b
                pltpu.make_async_copy(hp_hbm.at[src_v.at[j]], buf,
                                      sem).wait()
                pltpu.sync_copy(buf, acc.at[dst_v.at[j]], add=True)

                @pl.when(j + NB < HSTG)
                def _():
                    pltpu.async_copy(hp_hbm.at[src_v.at[j + NB]], buf, sem)

            return 0

        lax.fori_loop(0, HSTG // NB, body, 0)

        j = HSTG - 1
        pltpu.make_async_copy(hp_hbm.at[src_v.at[j]], buf0, sem0).wait()
        pltpu.sync_copy(buf0, acc.at[dst_v.at[j]], add=True)

    plsc.subcore_barrier()

    ocid = out_hbm.at[cid]
    for k in range(6):
        r0 = base + k * ECH
        pltpu.sync_copy(acc.at[pl.ds(r0, ECH)], buf0)
        pltpu.sync_copy(buf0, ocid.at[pl.ds(r0, ECH)])
    r0 = base + 600
    pltpu.sync_copy(acc.at[pl.ds(r0, 25)], buf0.at[pl.ds(0, 25)])
    pltpu.sync_copy(buf0.at[pl.ds(0, 25)], ocid.at[pl.ds(r0, 25)])


_edge_call = pl.kernel(
    _edge_body,
    out_type=jax.ShapeDtypeStruct((NC, N, D), jnp.float32),
    mesh=_mesh,
    compiler_params=pltpu.CompilerParams(use_tc_tiling_on_sc=False),
    scratch_types=[
        pltpu.VMEM((HSTG, ECH), jnp.int32),
        pltpu.VMEM((HSTG, ECH), jnp.int32),
        pltpu.VMEM((ECH, D), jnp.float32),
        pltpu.VMEM((ECH, D), jnp.float32),
        pltpu.VMEM((ECH, D), jnp.float32),
        pltpu.VMEM_SHARED((N, D), jnp.float32),
        pltpu.SemaphoreType.DMA,
        pltpu.SemaphoreType.DMA,
        pltpu.SemaphoreType.DMA,
    ],
)


def _k0_body(degp_ref, x_ref, w1_ref, dinv_ref, hp_ref):
    deg = jnp.sum(degp_ref[...], axis=0) + 1.0
    dinv = lax.rsqrt(deg)
    dinv_ref[...] = dinv
    h = jnp.dot(x_ref[...], w1_ref[...], preferred_element_type=jnp.float32)
    hp_ref[...] = h * dinv[:, None]


def _k0(deg_parts, x, W1):
    return pl.pallas_call(
        _k0_body,
        out_shape=[
            jax.ShapeDtypeStruct((N,), jnp.float32),
            jax.ShapeDtypeStruct((N, D), jnp.float32),
        ],
    )(deg_parts, x, W1)


def _fuse_mid_body(pacc_ref, hp_ref, dinv_ref, b_ref, g_ref, be_ref,
                   w_ref, out_ref):
    dinv = dinv_ref[...]
    s = pacc_ref[0] + pacc_ref[1] + hp_ref[...]
    agg = s * dinv[:, None] + b_ref[...][None, :]
    mean = jnp.mean(agg, axis=0)
    var = jnp.mean(agg * agg, axis=0) - mean * mean
    xn = (agg - mean[None, :]) * lax.rsqrt(var + 1e-5)[None, :]
    h = jnp.maximum(xn * g_ref[...][None, :] + be_ref[...][None, :], 0.0)
    hw = jnp.dot(h, w_ref[...], preferred_element_type=jnp.float32)
    out_ref[...] = hw * dinv[:, None]


def _fuse_mid(pacc, hp, dinv, b, g, be, W):
    return pl.pallas_call(
        _fuse_mid_body,
        out_shape=jax.ShapeDtypeStruct((N, D), jnp.float32),
    )(pacc, hp, dinv, b, g, be, W)


def _fuse_tail_body(pacc_ref, hp_ref, dinv_ref, b_ref, gw1_ref, gb1_ref,
                    gw2_ref, gb2_ref, batch_ref, mw1_ref, mb1_ref, mw2_ref,
                    mb2_ref, out_ref):
    s = pacc_ref[0] + pacc_ref[1] + hp_ref[...]
    h3 = s * dinv_ref[...][:, None] + b_ref[...][None, :]
    gmid = jnp.maximum(
        jnp.dot(h3, gw1_ref[...], preferred_element_type=jnp.float32)
        + gb1_ref[...][None, :], 0.0)
    gate = (jnp.dot(gmid, gw2_ref[...], preferred_element_type=jnp.float32)
            + gb2_ref[...][None, :])[:, 0]
    oh = batch_ref[...][:, None] == lax.broadcasted_iota(jnp.int32, (1, G), 1)
    mx = jnp.max(jnp.where(oh, gate[:, None], -jnp.inf), axis=0)
    mx = jnp.where(jnp.isfinite(mx), mx, 0.0)
    ohf = oh.astype(jnp.float32)
    mxb = jnp.dot(ohf, mx[:, None], preferred_element_type=jnp.float32)[:, 0]
    e = jnp.exp(gate - mxb)
    den = jnp.sum(ohf * e[:, None], axis=0)
    ew = lax.dot_general(ohf, h3 * e[:, None], (((0,), (0,)), ((), ())),
                         preferred_element_type=jnp.float32)
    pooled = ew / (den + 1e-16)[:, None]
    m1 = jnp.maximum(
        jnp.dot(pooled, mw1_ref[...], preferred_element_type=jnp.float32)
        + mb1_ref[...][None, :], 0.0)
    out_ref[...] = (jnp.dot(m1, mw2_ref[...],
                            preferred_element_type=jnp.float32)
                    + mb2_ref[...][None, :])


def _fuse_tail(pacc, hp, dinv, b, gW1, gb1, gW2, gb2, batch,
               mW1, mb1, mW2, mb2):
    return pl.pallas_call(
        _fuse_tail_body,
        out_shape=jax.ShapeDtypeStruct((G, D), jnp.float32),
    )(pacc, hp, dinv, b, gW1, gb1, gW2, gb2, batch, mW1, mb1, mW2, mb2)


def kernel(x, edge_index, batch, W1, b1, W2, b2, W3, b3, g1, be1, g2, be2,
           gW1, gb1, gW2, gb2, mW1, mb1, mW2, mb2):
    src3 = edge_index[0].reshape(NW, NCH2, ECH)
    dst3 = edge_index[1].reshape(NW, NCH2, ECH)
    dst3d = edge_index[1].reshape(NW, NCHUNK, CHUNK)

    deg_parts = _deg_call(dst3d)
    dinv, hp1 = _k0(deg_parts, x, W1)

    pacc1 = _edge_call(hp1, src3, dst3)
    hp2 = _fuse_mid(pacc1, hp1, dinv, b1, g1, be1, W2)

    pacc2 = _edge_call(hp2, src3, dst3)
    hp3 = _fuse_mid(pacc2, hp2, dinv, b2, g2, be2, W3)

    pacc3 = _edge_call(hp3, src3, dst3)
    return _fuse_tail(pacc3, hp3, dinv, b3, gW1, gb1, gW2, gb2, batch,
                      mW1, mb1, mW2, mb2)
